# Initial kernel scaffold; baseline (speedup 1.0000x reference)
import jax
import jax.numpy as jnp
from jax.experimental import pallas as pl


def kernel(in_feat, edge_index, W_self1, W_neigh1, b1, W_self2, W_neigh2, b2, W_self3, W_neigh3, b3):
    raise NotImplementedError("write your pallas kernel here")


# speedup vs baseline: 3.9023x; 3.9023x over previous
import jax
import jax.numpy as jnp
from jax import lax
from jax.experimental import pallas as pl
from jax.experimental.pallas import tpu as pltpu
from jax.experimental.pallas import tpu_sc as plsc

NC = 2
NS = 16
L = 16
NW = NC * NS
EC = 128


def _make_sc_agg(n_pad, d, e_pad, first):
    ep_agg = e_pad // NW
    n_chunk_agg = ep_agg // EC
    ep_deg = e_pad // NS
    n_chunk_deg = ep_deg // EC
    rows_tile = n_pad // NS
    n_row_chunks = rows_tile // EC

    mesh = plsc.VectorSubcoreMesh(core_axis_name="c", subcore_axis_name="s")
    out_type = [jax.ShapeDtypeStruct((NC, n_pad, d), jnp.float32)]
    scratch = [
        pltpu.VMEM((EC,), jnp.int32),
        pltpu.VMEM((EC,), jnp.int32),
        pltpu.VMEM((EC, d), jnp.float32),
        pltpu.VMEM((rows_tile,), jnp.float32),
        pltpu.VMEM((EC,), jnp.float32),
        pltpu.VMEM_SHARED((n_pad, d), jnp.float32),
        pltpu.SemaphoreType.DMA,
    ]
    if first:
        out_type.append(jax.ShapeDtypeStruct((n_pad,), jnp.float32))
        scratch.append(pltpu.VMEM_SHARED((n_pad,), jnp.float32))

    def body(h, src, dst, agg_out, deg_out,
             srcbuf, dstbuf, rowsbuf, degbuf, onesbuf, agg_sh, sem,
             deg_sh=None):
        c = lax.axis_index("c")
        s = lax.axis_index("s")
        wid = c * NS + s
        rbase = s * rows_tile
        zeros = jnp.zeros((L,), jnp.float32)

        def zrow(r, carry):
            for j in range(d // L):
                rowsbuf[r, pl.ds(j * L, L)] = zeros
            return carry
        lax.fori_loop(0, EC, zrow, 0)
        for j in range(EC // L):
            onesbuf[pl.ds(j * L, L)] = jnp.ones((L,), jnp.float32)

        for k in range(n_row_chunks):
            pltpu.sync_copy(rowsbuf, agg_sh.at[pl.ds(rbase + k * EC, EC), :])
        if first:
            def zr(i, carry):
                degbuf[pl.ds(i * L, L)] = zeros
                return carry
            lax.fori_loop(0, rows_tile // L, zr, 0)
            pltpu.sync_copy(degbuf, deg_sh.at[pl.ds(rbase, rows_tile)])
        plsc.subcore_barrier()

        if first:
            ebase_deg = s * ep_deg

            def degstep(k, carry):
                pltpu.sync_copy(dst.at[pl.ds(ebase_deg + k * EC, EC)], dstbuf)
                pltpu.sync_copy(onesbuf, deg_sh.at[dstbuf], add=True)
                return carry
            lax.fori_loop(0, n_chunk_deg, degstep, 0)

        ebase = wid * ep_agg

        def aggstep(k, carry):
            pltpu.sync_copy(src.at[pl.ds(ebase + k * EC, EC)], srcbuf)
            pltpu.sync_copy(dst.at[pl.ds(ebase + k * EC, EC)], dstbuf)
            pltpu.async_copy(h.at[srcbuf], rowsbuf, sem).wait()
            pltpu.sync_copy(rowsbuf, agg_sh.at[dstbuf], add=True)
            return carry
        lax.fori_loop(0, n_chunk_agg, aggstep, 0)
        plsc.subcore_barrier()

        pltpu.sync_copy(agg_sh.at[pl.ds(rbase, rows_tile), :],
                        agg_out.at[c, pl.ds(rbase, rows_tile), :])
        if first:
            @pl.when(c == 0)
            def _():
                pltpu.sync_copy(deg_sh.at[pl.ds(rbase, rows_tile)],
                                deg_out.at[pl.ds(rbase, rows_tile)])

    if first:
        def body_first(h, src, dst, agg_out, deg_out, *rest):
            return body(h, src, dst, agg_out, deg_out, *rest)
        fn = body_first
    else:
        def body_rest(h, src, dst, agg_out, *rest):
            return body(h, src, dst, agg_out, None, *rest)
        fn = body_rest

    return pl.kernel(fn, out_type=out_type, mesh=mesh, scratch_types=scratch)


def _make_tc_dense(n_pad, d, bsz):
    def tc_body(h_ref, m0_ref, m1_ref, deg_ref, ws_ref, wn_ref, b_ref, o_ref):
        recip = 1.0 / jnp.maximum(deg_ref[...], 1.0)
        mean = (m0_ref[...] + m1_ref[...]) * recip
        acc = jnp.dot(h_ref[...], ws_ref[...],
                      preferred_element_type=jnp.float32)
        acc = acc + jnp.dot(mean, wn_ref[...],
                            preferred_element_type=jnp.float32)
        o_ref[...] = jnp.maximum(acc + b_ref[...], 0.0)

    return pl.pallas_call(
        tc_body,
        grid=(n_pad // bsz,),
        in_specs=[
            pl.BlockSpec((bsz, d), lambda i: (i, 0)),
            pl.BlockSpec((bsz, d), lambda i: (i, 0)),
            pl.BlockSpec((bsz, d), lambda i: (i, 0)),
            pl.BlockSpec((bsz, 1), lambda i: (i, 0)),
            pl.BlockSpec((d, d), lambda i: (0, 0)),
            pl.BlockSpec((d, d), lambda i: (0, 0)),
            pl.BlockSpec((1, d), lambda i: (0, 0)),
        ],
        out_specs=pl.BlockSpec((bsz, d), lambda i: (i, 0)),
        out_shape=jax.ShapeDtypeStruct((n_pad, d), jnp.float32),
    )


def kernel(in_feat, edge_index, W_self1, W_neigh1, b1,
           W_self2, W_neigh2, b2, W_self3, W_neigh3, b3):
    n, d = in_feat.shape
    e = edge_index.shape[1]
    row_quant = NS * EC
    n_pad = ((n + row_quant - 1) // row_quant) * row_quant
    edge_quant = NW * EC
    e_pad = ((e + edge_quant - 1) // edge_quant) * edge_quant

    src = edge_index[0].astype(jnp.int32)
    dst = edge_index[1].astype(jnp.int32)
    src = jnp.concatenate([src, jnp.zeros((e_pad - e,), jnp.int32)])
    dst = jnp.concatenate([dst, jnp.full((e_pad - e,), n, jnp.int32)])
    x = jnp.pad(in_feat, ((0, n_pad - n), (0, 0)))

    sc_first = _make_sc_agg(n_pad, d, e_pad, first=True)
    sc_rest = _make_sc_agg(n_pad, d, e_pad, first=False)
    tc = _make_tc_dense(n_pad, d, bsz=512)

    m, deg = sc_first(x, src, dst)
    deg2 = deg.reshape(n_pad, 1)
    h = tc(x, m[0], m[1], deg2, W_self1, W_neigh1, b1.reshape(1, d))
    [m] = sc_rest(h, src, dst)
    h = tc(h, m[0], m[1], deg2, W_self2, W_neigh2, b2.reshape(1, d))
    [m] = sc_rest(h, src, dst)
    h = tc(h, m[0], m[1], deg2, W_self3, W_neigh3, b3.reshape(1, d))
    return h[:n]
